# baseline (speedup 1.0000x reference)
import functools

import jax
import jax.numpy as jnp
from jax import lax
from jax.experimental import pallas as pl
from jax.experimental.pallas import tpu as pltpu
from jax.experimental.pallas import tpu_sc as plsc

N = 10000
E = 320000
D = 128
H = 16
NC = 2
NS = 16
L = 16
NW = NC * NS
EW = E // NW
CH = 125
NCH = EW // CH
NB = 5
NP = NS * 640
STRIPE = NP // NS
PN = N // 8
PP = NP // 8


def _sc_mesh():
    return plsc.VectorSubcoreMesh(
        core_axis_name="c", subcore_axis_name="s", num_cores=NC, num_subcores=NS
    )


def _zero_rows(ref, n):
    def body(j, _):
        ref[j, :] = jnp.zeros((L,), jnp.float32)
        return 0
    lax.fori_loop(0, n, body, 0)


def _deg_body(e3, out, dbuf, ones_v, obuf, acc, *sems):
    c = lax.axis_index("c")
    s = lax.axis_index("s")
    wid = s * NC + c
    pltpu.async_copy(e3.at[1, wid], dbuf, sems[0])

    def fill(j, _):
        ones_v[j, :] = jnp.ones((L,), jnp.float32)
        return 0
    lax.fori_loop(0, CH, fill, 0)

    _zero_rows(obuf, STRIPE)
    pltpu.sync_copy(obuf, acc.at[pl.ds(s * STRIPE, STRIPE)])
    pltpu.make_async_copy(e3.at[1, wid], dbuf, sems[0]).wait()
    plsc.subcore_barrier()

    for b in range(NB):
        pltpu.async_copy(ones_v, acc.at[dbuf.at[b]], sems[b], add=True)

    def step(j, _):
        k0 = NB * j
        for b in range(NB):
            pltpu.make_async_copy(ones_v, acc.at[dbuf.at[k0 - NB + b]], sems[b]).wait()
            pltpu.async_copy(ones_v, acc.at[dbuf.at[k0 + b]], sems[b], add=True)
        return 0
    lax.fori_loop(1, NCH // NB, step, 0)

    for b in range(NB):
        pltpu.make_async_copy(ones_v, acc.at[dbuf.at[NCH - NB + b]], sems[b]).wait()

    plsc.subcore_barrier()
    pltpu.sync_copy(acc.at[pl.ds(s * STRIPE, STRIPE)], obuf)
    pltpu.sync_copy(obuf, out.at[c, pl.ds(s * STRIPE, STRIPE)])


def _agg_body(e3, table, out, sbuf, dbuf, rows, obuf, acc, tbl, stg, *sems):
    c = lax.axis_index("c")
    s = lax.axis_index("s")
    wid = s * NC + c
    gs = sems[:NB]
    ts = sems[NB:]
    pltpu.async_copy(e3.at[0, wid], sbuf, gs[0])
    pltpu.async_copy(e3.at[1, wid], dbuf, gs[1])
    pltpu.async_copy(table.at[pl.ds(s * STRIPE, STRIPE)], stg, gs[2])
    _zero_rows(obuf, STRIPE)
    pltpu.make_async_copy(table.at[pl.ds(s * STRIPE, STRIPE)], stg, gs[2]).wait()
    pltpu.sync_copy(stg, tbl.at[pl.ds(s * STRIPE, STRIPE)])
    pltpu.sync_copy(obuf, acc.at[pl.ds(s * STRIPE, STRIPE)])
    pltpu.make_async_copy(e3.at[0, wid], sbuf, gs[0]).wait()
    pltpu.make_async_copy(e3.at[1, wid], dbuf, gs[1]).wait()
    plsc.subcore_barrier()

    def gather(k, b):
        pltpu.async_copy(tbl.at[sbuf.at[k]], rows.at[b], gs[b])

    def gwait(k, b):
        pltpu.make_async_copy(tbl.at[sbuf.at[k]], rows.at[b], gs[b]).wait()

    def scat(k, b):
        pltpu.async_copy(rows.at[b], acc.at[dbuf.at[k]], ts[b], add=True)

    def swait(k, b):
        pltpu.make_async_copy(rows.at[b], acc.at[dbuf.at[k]], ts[b]).wait()

    for b in range(NB):
        gather(b, b)

    def step(j, _):
        k0 = NB * j
        gwait(k0 + 0, 0); scat(k0 + 0, 0)
        gwait(k0 + 1, 1); scat(k0 + 1, 1)
        gwait(k0 + 2, 2); scat(k0 + 2, 2)
        swait(k0 + 0, 0); gather(k0 + 5, 0)
        gwait(k0 + 3, 3); scat(k0 + 3, 3)
        swait(k0 + 1, 1); gather(k0 + 6, 1)
        gwait(k0 + 4, 4); scat(k0 + 4, 4)
        swait(k0 + 2, 2); gather(k0 + 7, 2)
        swait(k0 + 3, 3); gather(k0 + 8, 3)
        swait(k0 + 4, 4); gather(k0 + 9, 4)
        return 0
    lax.fori_loop(0, NCH // NB - 1, step, 0)

    k0 = NCH - NB
    gwait(k0 + 0, 0); scat(k0 + 0, 0)
    gwait(k0 + 1, 1); scat(k0 + 1, 1)
    swait(k0 + 0, 0)
    gwait(k0 + 2, 2); scat(k0 + 2, 2)
    swait(k0 + 1, 1)
    gwait(k0 + 3, 3); scat(k0 + 3, 3)
    swait(k0 + 2, 2)
    gwait(k0 + 4, 4); scat(k0 + 4, 4)
    swait(k0 + 3, 3)
    swait(k0 + 4, 4)

    plsc.subcore_barrier()
    pltpu.sync_copy(acc.at[pl.ds(s * STRIPE, STRIPE)], obuf)
    pltpu.sync_copy(obuf, out.at[c, pl.ds(s * STRIPE, STRIPE)])


@functools.partial(
    pl.kernel,
    out_type=jax.ShapeDtypeStruct((NC, NP, H), jnp.float32),
    mesh=_sc_mesh(),
    compiler_params=pltpu.CompilerParams(use_tc_tiling_on_sc=False),
    scratch_types=[
        pltpu.VMEM((NCH, CH), jnp.int32),
        pltpu.VMEM((CH, H), jnp.float32),
        pltpu.VMEM((STRIPE, H), jnp.float32),
        pltpu.VMEM_SHARED((NP, H), jnp.float32),
    ] + [pltpu.SemaphoreType.DMA] * NB,
)
def _deg_call(e3, out, dbuf, ones_v, obuf, acc, *sems):
    _deg_body(e3, out, dbuf, ones_v, obuf, acc, *sems)


@functools.partial(
    pl.kernel,
    out_type=jax.ShapeDtypeStruct((NC, NP, H), jnp.float32),
    mesh=_sc_mesh(),
    compiler_params=pltpu.CompilerParams(use_tc_tiling_on_sc=False),
    scratch_types=[
        pltpu.VMEM((NCH, CH), jnp.int32),
        pltpu.VMEM((NCH, CH), jnp.int32),
        pltpu.VMEM((NB, CH, H), jnp.float32),
        pltpu.VMEM((STRIPE, H), jnp.float32),
        pltpu.VMEM_SHARED((NP, H), jnp.float32),
        pltpu.VMEM_SHARED((NP, H), jnp.float32),
        pltpu.VMEM((STRIPE, H), jnp.float32),
    ] + [pltpu.SemaphoreType.DMA] * (2 * NB),
)
def _agg_call(e3, table, out, sbuf, dbuf, rows, obuf, acc, tbl, stg, *sems):
    _agg_body(e3, table, out, sbuf, dbuf, rows, obuf, acc, tbl, stg, *sems)


def _hs_body(x_ref, w1_ref, p_ref, hs_ref, d_ref):
    d = lax.rsqrt(1.0 + p_ref[0] + p_ref[1])
    h = jnp.dot(x_ref[...], w1_ref[...], preferred_element_type=jnp.float32)
    hs = h * d[:PN]
    hs_ref[...] = jnp.concatenate(
        [hs, jnp.zeros((PP - PN, D), jnp.float32)], axis=0)
    d_ref[...] = d


def _mid_body(s_ref, hs_ref, d_ref, b1_ref, o_ref):
    d = d_ref[...]
    agg = d * (s_ref[0] + s_ref[1] + hs_ref[...]) + b1_ref[...]
    o_ref[...] = jnp.maximum(agg, 0.0) * d


def _out_body(s_ref, hs2_ref, d_ref, w2_ref, b2_ref, o_ref):
    agg = d_ref[...] * (s_ref[0] + s_ref[1] + hs2_ref[...])
    o_ref[...] = (
        jnp.dot(agg, w2_ref[...], preferred_element_type=jnp.float32)
        + b2_ref[...]
    )


def _full(*dims):
    return pl.BlockSpec(dims, lambda: tuple(0 for _ in dims))


_hs_call = pl.pallas_call(
    _hs_body,
    in_specs=[
        _full(PN, 8 * D),
        _full(8 * D, D),
        _full(NC, PP, D),
    ],
    out_specs=[_full(PP, D), _full(PP, D)],
    out_shape=[
        jax.ShapeDtypeStruct((PP, D), jnp.float32),
        jax.ShapeDtypeStruct((PP, D), jnp.float32),
    ],
)

_mid_call = pl.pallas_call(
    _mid_body,
    in_specs=[
        _full(NC, PP, D),
        _full(PP, D),
        _full(PP, D),
        _full(1, D),
    ],
    out_specs=_full(PP, D),
    out_shape=jax.ShapeDtypeStruct((PP, D), jnp.float32),
)

_out_call = pl.pallas_call(
    _out_body,
    in_specs=[
        _full(NC, PP, D),
        _full(PP, D),
        _full(PP, D),
        _full(D, 8 * D),
        _full(1, 8 * D),
    ],
    out_specs=_full(PP, 8 * D),
    out_shape=jax.ShapeDtypeStruct((PP, 8 * D), jnp.float32),
)


def kernel(x, edge_index, W1, b1, W2, b2):
    e3 = edge_index.reshape(2, NW, NCH, CH)
    xr = x.reshape(PN, 8 * D)
    w1e = jnp.kron(jnp.eye(8, dtype=jnp.float32), W1)
    w2e = jnp.kron(jnp.eye(8, dtype=jnp.float32), W2)
    b1t = jnp.tile(b1, 8).reshape(1, D)
    b2t = jnp.tile(b2, 8).reshape(1, 8 * D)
    degp = _deg_call(e3).reshape(NC, PP, D)
    hs_pk, d_pk = _hs_call(xr, w1e, degp)
    s1 = _agg_call(e3, hs_pk.reshape(NP, H)).reshape(NC, PP, D)
    hs2_pk = _mid_call(s1, hs_pk, d_pk, b1t)
    s2 = _agg_call(e3, hs2_pk.reshape(NP, H)).reshape(NC, PP, D)
    out_pk = _out_call(s2, hs2_pk, d_pk, w2e, b2t)
    return out_pk.reshape(NP, D)[:N]


# speedup vs baseline: 1.0226x; 1.0226x over previous
import functools

import jax
import jax.numpy as jnp
from jax import lax
from jax.experimental import pallas as pl
from jax.experimental.pallas import tpu as pltpu
from jax.experimental.pallas import tpu_sc as plsc

N = 10000
E = 320000
D = 128
H = 16
NC = 2
NS = 16
L = 16
NW = NC * NS
EW = E // NW
CH = 125
NCH = EW // CH
NB = 5
NP = NS * 640
STRIPE = NP // NS
PN = N // 8
PP = NP // 8


def _sc_mesh():
    return plsc.VectorSubcoreMesh(
        core_axis_name="c", subcore_axis_name="s", num_cores=NC, num_subcores=NS
    )


def _zero_rows(ref, n):
    def body(j, _):
        ref[j, :] = jnp.zeros((L,), jnp.float32)
        return 0
    lax.fori_loop(0, n, body, 0)


def _deg_body(e3, out, dbuf, ones_v, obuf, acc, *sems):
    c = lax.axis_index("c")
    s = lax.axis_index("s")
    wid = s * NC + c
    pltpu.async_copy(e3.at[1, wid], dbuf, sems[0])

    def fill(j, _):
        ones_v[j, :] = jnp.ones((L,), jnp.float32)
        return 0
    lax.fori_loop(0, CH, fill, 0)

    _zero_rows(obuf, STRIPE)
    pltpu.sync_copy(obuf, acc.at[pl.ds(s * STRIPE, STRIPE)])
    pltpu.make_async_copy(e3.at[1, wid], dbuf, sems[0]).wait()
    plsc.subcore_barrier()

    for b in range(NB):
        pltpu.async_copy(ones_v, acc.at[dbuf.at[b]], sems[b], add=True)

    def step(j, _):
        k0 = NB * j
        for b in range(NB):
            pltpu.make_async_copy(ones_v, acc.at[dbuf.at[k0 - NB + b]], sems[b]).wait()
            pltpu.async_copy(ones_v, acc.at[dbuf.at[k0 + b]], sems[b], add=True)
        return 0
    lax.fori_loop(1, NCH // NB, step, 0)

    for b in range(NB):
        pltpu.make_async_copy(ones_v, acc.at[dbuf.at[NCH - NB + b]], sems[b]).wait()

    plsc.subcore_barrier()
    pltpu.sync_copy(acc.at[pl.ds(s * STRIPE, STRIPE)], obuf)
    pltpu.sync_copy(obuf, out.at[c, pl.ds(s * STRIPE, STRIPE)])


def _agg_body(e3, table, out, sbuf, dbuf, rows, obuf, acc, tbl, stg, *sems):
    c = lax.axis_index("c")
    s = lax.axis_index("s")
    wid = s * NC + c
    gs = sems[:NB]
    ts = sems[NB:]
    pltpu.async_copy(e3.at[0, wid], sbuf, gs[0])
    pltpu.async_copy(e3.at[1, wid], dbuf, gs[1])
    pltpu.async_copy(table.at[pl.ds(s * STRIPE, STRIPE)], stg, gs[2])
    _zero_rows(obuf, STRIPE)
    pltpu.make_async_copy(table.at[pl.ds(s * STRIPE, STRIPE)], stg, gs[2]).wait()
    pltpu.sync_copy(stg, tbl.at[pl.ds(s * STRIPE, STRIPE)])
    pltpu.sync_copy(obuf, acc.at[pl.ds(s * STRIPE, STRIPE)])
    pltpu.make_async_copy(e3.at[0, wid], sbuf, gs[0]).wait()
    pltpu.make_async_copy(e3.at[1, wid], dbuf, gs[1]).wait()
    plsc.subcore_barrier()

    def gather(k, b):
        pltpu.async_copy(tbl.at[sbuf.at[k]], rows.at[b], gs[b])

    def gwait(k, b):
        pltpu.make_async_copy(tbl.at[sbuf.at[k]], rows.at[b], gs[b]).wait()

    def scat(k, b):
        pltpu.async_copy(rows.at[b], acc.at[dbuf.at[k]], ts[b], add=True)

    def swait(k, b):
        pltpu.make_async_copy(rows.at[b], acc.at[dbuf.at[k]], ts[b]).wait()

    for b in range(NB):
        gather(b, b)

    def step(j, _):
        k0 = NB * j
        gwait(k0 + 0, 0); scat(k0 + 0, 0)
        gwait(k0 + 1, 1); scat(k0 + 1, 1)
        swait(k0 + 0, 0); gather(k0 + 5, 0)
        gwait(k0 + 2, 2); scat(k0 + 2, 2)
        swait(k0 + 1, 1); gather(k0 + 6, 1)
        gwait(k0 + 3, 3); scat(k0 + 3, 3)
        swait(k0 + 2, 2); gather(k0 + 7, 2)
        gwait(k0 + 4, 4); scat(k0 + 4, 4)
        swait(k0 + 3, 3); gather(k0 + 8, 3)
        swait(k0 + 4, 4); gather(k0 + 9, 4)
        return 0
    lax.fori_loop(0, NCH // NB - 1, step, 0)

    k0 = NCH - NB
    gwait(k0 + 0, 0); scat(k0 + 0, 0)
    gwait(k0 + 1, 1); scat(k0 + 1, 1)
    swait(k0 + 0, 0)
    gwait(k0 + 2, 2); scat(k0 + 2, 2)
    swait(k0 + 1, 1)
    gwait(k0 + 3, 3); scat(k0 + 3, 3)
    swait(k0 + 2, 2)
    gwait(k0 + 4, 4); scat(k0 + 4, 4)
    swait(k0 + 3, 3)
    swait(k0 + 4, 4)

    plsc.subcore_barrier()
    pltpu.sync_copy(acc.at[pl.ds(s * STRIPE, STRIPE)], obuf)
    pltpu.sync_copy(obuf, out.at[c, pl.ds(s * STRIPE, STRIPE)])


@functools.partial(
    pl.kernel,
    out_type=jax.ShapeDtypeStruct((NC, NP, H), jnp.float32),
    mesh=_sc_mesh(),
    compiler_params=pltpu.CompilerParams(use_tc_tiling_on_sc=False),
    scratch_types=[
        pltpu.VMEM((NCH, CH), jnp.int32),
        pltpu.VMEM((CH, H), jnp.float32),
        pltpu.VMEM((STRIPE, H), jnp.float32),
        pltpu.VMEM_SHARED((NP, H), jnp.float32),
    ] + [pltpu.SemaphoreType.DMA] * NB,
)
def _deg_call(e3, out, dbuf, ones_v, obuf, acc, *sems):
    _deg_body(e3, out, dbuf, ones_v, obuf, acc, *sems)


@functools.partial(
    pl.kernel,
    out_type=jax.ShapeDtypeStruct((NC, NP, H), jnp.float32),
    mesh=_sc_mesh(),
    compiler_params=pltpu.CompilerParams(use_tc_tiling_on_sc=False),
    scratch_types=[
        pltpu.VMEM((NCH, CH), jnp.int32),
        pltpu.VMEM((NCH, CH), jnp.int32),
        pltpu.VMEM((NB, CH, H), jnp.float32),
        pltpu.VMEM((STRIPE, H), jnp.float32),
        pltpu.VMEM_SHARED((NP, H), jnp.float32),
        pltpu.VMEM_SHARED((NP, H), jnp.float32),
        pltpu.VMEM((STRIPE, H), jnp.float32),
    ] + [pltpu.SemaphoreType.DMA] * (2 * NB),
)
def _agg_call(e3, table, out, sbuf, dbuf, rows, obuf, acc, tbl, stg, *sems):
    _agg_body(e3, table, out, sbuf, dbuf, rows, obuf, acc, tbl, stg, *sems)


def _hs_body(x_ref, w1_ref, p_ref, hs_ref, d_ref):
    d = lax.rsqrt(1.0 + p_ref[0] + p_ref[1])
    h = jnp.dot(x_ref[...], w1_ref[...], preferred_element_type=jnp.float32)
    hs = h * d[:PN]
    hs_ref[...] = jnp.concatenate(
        [hs, jnp.zeros((PP - PN, D), jnp.float32)], axis=0)
    d_ref[...] = d


def _mid_body(s_ref, hs_ref, d_ref, b1_ref, o_ref):
    d = d_ref[...]
    agg = d * (s_ref[0] + s_ref[1] + hs_ref[...]) + b1_ref[...]
    o_ref[...] = jnp.maximum(agg, 0.0) * d


def _out_body(s_ref, hs2_ref, d_ref, w2_ref, b2_ref, o_ref):
    agg = d_ref[...] * (s_ref[0] + s_ref[1] + hs2_ref[...])
    o_ref[...] = (
        jnp.dot(agg, w2_ref[...], preferred_element_type=jnp.float32)
        + b2_ref[...]
    )


def _full(*dims):
    return pl.BlockSpec(dims, lambda: tuple(0 for _ in dims))


_hs_call = pl.pallas_call(
    _hs_body,
    in_specs=[
        _full(PN, 8 * D),
        _full(8 * D, D),
        _full(NC, PP, D),
    ],
    out_specs=[_full(PP, D), _full(PP, D)],
    out_shape=[
        jax.ShapeDtypeStruct((PP, D), jnp.float32),
        jax.ShapeDtypeStruct((PP, D), jnp.float32),
    ],
)

_mid_call = pl.pallas_call(
    _mid_body,
    in_specs=[
        _full(NC, PP, D),
        _full(PP, D),
        _full(PP, D),
        _full(1, D),
    ],
    out_specs=_full(PP, D),
    out_shape=jax.ShapeDtypeStruct((PP, D), jnp.float32),
)

_out_call = pl.pallas_call(
    _out_body,
    in_specs=[
        _full(NC, PP, D),
        _full(PP, D),
        _full(PP, D),
        _full(D, 8 * D),
        _full(1, 8 * D),
    ],
    out_specs=_full(PP, 8 * D),
    out_shape=jax.ShapeDtypeStruct((PP, 8 * D), jnp.float32),
)


def kernel(x, edge_index, W1, b1, W2, b2):
    e3 = edge_index.reshape(2, NW, NCH, CH)
    xr = x.reshape(PN, 8 * D)
    w1e = jnp.kron(jnp.eye(8, dtype=jnp.float32), W1)
    w2e = jnp.kron(jnp.eye(8, dtype=jnp.float32), W2)
    b1t = jnp.tile(b1, 8).reshape(1, D)
    b2t = jnp.tile(b2, 8).reshape(1, 8 * D)
    degp = _deg_call(e3).reshape(NC, PP, D)
    hs_pk, d_pk = _hs_call(xr, w1e, degp)
    s1 = _agg_call(e3, hs_pk.reshape(NP, H)).reshape(NC, PP, D)
    hs2_pk = _mid_call(s1, hs_pk, d_pk, b1t)
    s2 = _agg_call(e3, hs2_pk.reshape(NP, H)).reshape(NC, PP, D)
    out_pk = _out_call(s2, hs2_pk, d_pk, w2e, b2t)
    return out_pk.reshape(NP, D)[:N]
